# baseline (speedup 1.0000x reference)
import jax
import jax.numpy as jnp
from jax.experimental import pallas as pl
from jax.experimental.pallas import tpu as pltpu

_CHUNK = 1000
_NB = 12
_K = 6
_NSEL = 64
_T = 8


def _body(x_hbm, o_hbm, bufs, insem, outsem):
    rows, cols = x_hbm.shape
    nchunks = rows // _CHUNK

    def in_copy(c):
        return pltpu.make_async_copy(
            x_hbm.at[pl.ds(c * _CHUNK, _CHUNK), :],
            bufs.at[c % _NB],
            insem.at[c % _NB],
        )

    def out_copy(c):
        return pltpu.make_async_copy(
            bufs.at[c % _NB],
            o_hbm.at[pl.ds(c * _CHUNK, _CHUNK), :],
            outsem.at[c % _NB],
        )

    for c in range(_K):
        in_copy(c).start()
    for c in range(nchunks):
        in_copy(c).wait()
        if c < _NSEL:
            b = c % _NB
            slab = bufs[b, 0:_T, :]
            rid = jax.lax.broadcasted_iota(jnp.int32, slab.shape, 0)
            bufs[b, 0:_T, :] = jnp.where(
                rid == 0, jnp.maximum(slab, 0.0), slab
            )
        out_copy(c).start()
        nxt = c + _K
        if nxt < nchunks:
            prev = nxt - _NB
            if prev >= 0:
                out_copy(prev).wait()
            in_copy(nxt).start()
    for c in range(max(nchunks - _NB, 0), nchunks):
        out_copy(c).wait()


def kernel(x):
    rows, cols = x.shape
    return pl.pallas_call(
        _body,
        in_specs=[pl.BlockSpec(memory_space=pltpu.HBM)],
        out_specs=pl.BlockSpec(memory_space=pltpu.HBM),
        out_shape=jax.ShapeDtypeStruct(x.shape, x.dtype),
        scratch_shapes=[
            pltpu.VMEM((_NB, _CHUNK, cols), jnp.float32),
            pltpu.SemaphoreType.DMA((_NB,)),
            pltpu.SemaphoreType.DMA((_NB,)),
        ],
        compiler_params=pltpu.CompilerParams(vmem_limit_bytes=100 * 1024 * 1024),
    )(x)
